# Initial kernel scaffold; baseline (speedup 1.0000x reference)
import jax
import jax.numpy as jnp
from jax.experimental import pallas as pl


def kernel(x, Win, emb_w, wc0_w, wc0_b, wc1_w, wc1_b, wc2_w, wc2_b, gc0_w, gc0_b, fc_w, fc_b, g_size):
    raise NotImplementedError("write your pallas kernel here")


# speedup vs baseline: 2.0152x; 2.0152x over previous
import jax
import jax.numpy as jnp
from jax.experimental import pallas as pl

_N = 2048
_F = 128
_BLK = 256
_NBLK = _N // _BLK


def _proj_body(x_ref, emb_ref, w0_ref, b0_ref, w1_ref, b1_ref, w2_ref, b2_ref,
               h_ref, h0_ref, h1_ref, h2_ref):
    h = jnp.dot(x_ref[...], emb_ref[...].T, preferred_element_type=jnp.float32)
    h_ref[...] = h
    h0_ref[...] = jnp.dot(h, w0_ref[...].T,
                          preferred_element_type=jnp.float32) + b0_ref[...]
    h1_ref[...] = jnp.dot(h, w1_ref[...].T,
                          preferred_element_type=jnp.float32) + b1_ref[...]
    h2_ref[...] = jnp.dot(h, w2_ref[...].T,
                          preferred_element_type=jnp.float32) + b2_ref[...]


def _main_body(h_ref, h0_ref, h1_ref, h2_ref, win_ref,
               gc_ref, gcb_ref, fc_ref, fcb_ref, out_ref):
    i = pl.program_id(0)
    row = pl.ds(i * _BLK, _BLK)
    h = h_ref[...]
    hb = h_ref[row, :]
    w1 = win_ref[...]
    w2 = w1 * w1
    w3 = w2 * w1
    scale = jnp.float32(1.0) / jnp.sqrt(jnp.float32(_F))

    acc = jnp.dot(hb, gc_ref[:, 0:_F].T, preferred_element_type=jnp.float32)

    col = 1
    for wk, hk_ref in ((w1, h0_ref), (w2, h1_ref), (w3, h2_ref)):
        hkb = hk_ref[row, :]
        s = jax.nn.sigmoid(
            jnp.dot(hkb, hk_ref[...].T, preferred_element_type=jnp.float32)
            * scale)
        m = jnp.dot(wk * s, h, preferred_element_type=jnp.float32)
        acc = acc + jnp.dot(m, gc_ref[:, col * _F:(col + 1) * _F].T,
                            preferred_element_type=jnp.float32)
        m = jnp.dot(wk * (s * s), h, preferred_element_type=jnp.float32)
        acc = acc + jnp.dot(m, gc_ref[:, (col + 1) * _F:(col + 2) * _F].T,
                            preferred_element_type=jnp.float32)
        col += 2

    hc = jnp.maximum(acc + gcb_ref[...], jnp.float32(0.0))
    out_ref[...] = jnp.dot(hc, fc_ref[...].T,
                           preferred_element_type=jnp.float32) + fcb_ref[...]


def _full(shape):
    nd = len(shape)
    return pl.BlockSpec(shape, lambda i: (0,) * nd)


def kernel(x, Win, emb_w, wc0_w, wc0_b, wc1_w, wc1_b, wc2_w, wc2_b,
           gc0_w, gc0_b, fc_w, fc_b, g_size):
    b0 = wc0_b.reshape(1, _F)
    b1 = wc1_b.reshape(1, _F)
    b2 = wc2_b.reshape(1, _F)
    gcb = gc0_b.reshape(1, _F)
    fcb = fc_b.reshape(1, _F)

    hF = jax.ShapeDtypeStruct((_N, _F), jnp.float32)
    h, h0, h1, h2 = pl.pallas_call(
        _proj_body,
        grid=(_NBLK,),
        in_specs=[
            pl.BlockSpec((_BLK, _F), lambda i: (i, 0)),
            _full((_F, _F)),
            _full((_F, _F)), _full((1, _F)),
            _full((_F, _F)), _full((1, _F)),
            _full((_F, _F)), _full((1, _F)),
        ],
        out_specs=[pl.BlockSpec((_BLK, _F), lambda i: (i, 0))] * 4,
        out_shape=[hF, hF, hF, hF],
    )(x, emb_w, wc0_w, b0, wc1_w, b1, wc2_w, b2)

    out = pl.pallas_call(
        _main_body,
        grid=(_NBLK,),
        in_specs=[
            _full((_N, _F)), _full((_N, _F)), _full((_N, _F)), _full((_N, _F)),
            pl.BlockSpec((_BLK, _N), lambda i: (i, 0)),
            _full((_F, 7 * _F)), _full((1, _F)),
            _full((_F, _F)), _full((1, _F)),
        ],
        out_specs=pl.BlockSpec((_BLK, _F), lambda i: (i, 0)),
        out_shape=jax.ShapeDtypeStruct((_N, _F), jnp.float32),
    )(h, h0, h1, h2, Win, gc0_w, gcb, fc_w, fcb)

    return (out, Win, g_size)


# speedup vs baseline: 2.0195x; 1.0021x over previous
import jax
import jax.numpy as jnp
from jax.experimental import pallas as pl
from jax.experimental.pallas import tpu as pltpu

_N = 2048
_F = 128
_BLK = 256
_NBLK = _N // _BLK


def _proj_body(x_ref, emb_ref, w0_ref, b0_ref, w1_ref, b1_ref, w2_ref, b2_ref,
               h_ref, h0_ref, h1_ref, h2_ref):
    h = jnp.dot(x_ref[...], emb_ref[...].T, preferred_element_type=jnp.float32)
    h_ref[...] = h
    h0_ref[...] = jnp.dot(h, w0_ref[...].T,
                          preferred_element_type=jnp.float32) + b0_ref[...]
    h1_ref[...] = jnp.dot(h, w1_ref[...].T,
                          preferred_element_type=jnp.float32) + b1_ref[...]
    h2_ref[...] = jnp.dot(h, w2_ref[...].T,
                          preferred_element_type=jnp.float32) + b2_ref[...]


def _main_body(h_ref, h0_ref, h1_ref, h2_ref, win_ref,
               gc_ref, gcb_ref, fc_ref, fcb_ref, out_ref):
    i = pl.program_id(0)
    row = pl.ds(i * _BLK, _BLK)
    h = h_ref[...]
    hb = h_ref[row, :]
    w1 = win_ref[...]
    w2 = w1 * w1
    w3 = w2 * w1
    scale = jnp.float32(1.0) / jnp.sqrt(jnp.float32(_F))

    acc = jnp.dot(hb, gc_ref[:, 0:_F].T, preferred_element_type=jnp.float32)

    col = 1
    for wk, hk_ref in ((w1, h0_ref), (w2, h1_ref), (w3, h2_ref)):
        hkb = hk_ref[row, :]
        s = jax.nn.sigmoid(
            jnp.dot(hkb, hk_ref[...].T, preferred_element_type=jnp.float32)
            * scale)
        m = jnp.dot(wk * s, h, preferred_element_type=jnp.float32)
        acc = acc + jnp.dot(m, gc_ref[:, col * _F:(col + 1) * _F].T,
                            preferred_element_type=jnp.float32)
        m = jnp.dot(wk * (s * s), h, preferred_element_type=jnp.float32)
        acc = acc + jnp.dot(m, gc_ref[:, (col + 1) * _F:(col + 2) * _F].T,
                            preferred_element_type=jnp.float32)
        col += 2

    hc = jnp.maximum(acc + gcb_ref[...], jnp.float32(0.0))
    out_ref[...] = jnp.dot(hc, fc_ref[...].T,
                           preferred_element_type=jnp.float32) + fcb_ref[...]


def _full(shape):
    nd = len(shape)
    return pl.BlockSpec(shape, lambda i: (0,) * nd)


def kernel(x, Win, emb_w, wc0_w, wc0_b, wc1_w, wc1_b, wc2_w, wc2_b,
           gc0_w, gc0_b, fc_w, fc_b, g_size):
    b0 = wc0_b.reshape(1, _F)
    b1 = wc1_b.reshape(1, _F)
    b2 = wc2_b.reshape(1, _F)
    gcb = gc0_b.reshape(1, _F)
    fcb = fc_b.reshape(1, _F)

    hF = jax.ShapeDtypeStruct((_N, _F), jnp.float32)
    h, h0, h1, h2 = pl.pallas_call(
        _proj_body,
        grid=(_NBLK,),
        in_specs=[
            pl.BlockSpec((_BLK, _F), lambda i: (i, 0)),
            _full((_F, _F)),
            _full((_F, _F)), _full((1, _F)),
            _full((_F, _F)), _full((1, _F)),
            _full((_F, _F)), _full((1, _F)),
        ],
        out_specs=[pl.BlockSpec((_BLK, _F), lambda i: (i, 0))] * 4,
        out_shape=[hF, hF, hF, hF],
        compiler_params=pltpu.CompilerParams(
            dimension_semantics=("parallel",)),
    )(x, emb_w, wc0_w, b0, wc1_w, b1, wc2_w, b2)

    out = pl.pallas_call(
        _main_body,
        grid=(_NBLK,),
        in_specs=[
            _full((_N, _F)), _full((_N, _F)), _full((_N, _F)), _full((_N, _F)),
            pl.BlockSpec((_BLK, _N), lambda i: (i, 0)),
            _full((_F, 7 * _F)), _full((1, _F)),
            _full((_F, _F)), _full((1, _F)),
        ],
        out_specs=pl.BlockSpec((_BLK, _F), lambda i: (i, 0)),
        out_shape=jax.ShapeDtypeStruct((_N, _F), jnp.float32),
        compiler_params=pltpu.CompilerParams(
            dimension_semantics=("parallel",)),
    )(h, h0, h1, h2, Win, gc0_w, gcb, fc_w, fcb)

    return (out, Win, g_size)


# speedup vs baseline: 2.0853x; 1.0326x over previous
import jax
import jax.numpy as jnp
from jax.experimental import pallas as pl
from jax.experimental.pallas import tpu as pltpu

_N = 2048
_F = 128
_BLK = 256
_NBLK = _N // _BLK


def _proj_body(x_ref, emb_ref, w0_ref, b0_ref, w1_ref, b1_ref, w2_ref, b2_ref,
               gc_ref, h0_ref, h1_ref, h2_ref, h0t_ref, h1t_ref, h2t_ref,
               hg0_ref, hg1_ref, hg2_ref, hg3_ref, hg4_ref, hg5_ref, hg6_ref):
    h = jnp.dot(x_ref[...], emb_ref[...].T, preferred_element_type=jnp.float32)
    for k, (w_ref, b_ref, hk_ref, hkt_ref) in enumerate((
            (w0_ref, b0_ref, h0_ref, h0t_ref),
            (w1_ref, b1_ref, h1_ref, h1t_ref),
            (w2_ref, b2_ref, h2_ref, h2t_ref))):
        hk = jnp.dot(h, w_ref[...].T,
                     preferred_element_type=jnp.float32) + b_ref[...]
        hk_ref[...] = hk
        hkt_ref[...] = hk.T
    for c, hg_ref in enumerate((hg0_ref, hg1_ref, hg2_ref, hg3_ref,
                                hg4_ref, hg5_ref, hg6_ref)):
        hg_ref[...] = jnp.dot(h, gc_ref[:, c * _F:(c + 1) * _F].T,
                              preferred_element_type=jnp.float32)


def _main_body(h0_ref, h1_ref, h2_ref, h0t_ref, h1t_ref, h2t_ref,
               hg0_ref, hg1_ref, hg2_ref, hg3_ref, hg4_ref, hg5_ref, hg6_ref,
               win_ref, gcb_ref, fc_ref, fcb_ref, out_ref):
    i = pl.program_id(0)
    row = pl.ds(i * _BLK, _BLK)
    w1 = win_ref[...]
    w2 = w1 * w1
    w3 = w2 * w1
    scale = jnp.float32(1.0) / jnp.sqrt(jnp.float32(_F))

    acc = hg0_ref[row, :]

    for hk_ref, hkt_ref, wk, hga_ref, hgb_ref in (
            (h0_ref, h0t_ref, w1, hg1_ref, hg2_ref),
            (h1_ref, h1t_ref, w2, hg3_ref, hg4_ref),
            (h2_ref, h2t_ref, w3, hg5_ref, hg6_ref)):
        s = jax.nn.sigmoid(
            jnp.dot(hk_ref[row, :], hkt_ref[...],
                    preferred_element_type=jnp.float32) * scale)
        acc = acc + jnp.dot(wk * s, hga_ref[...],
                            preferred_element_type=jnp.float32)
        acc = acc + jnp.dot(wk * (s * s), hgb_ref[...],
                            preferred_element_type=jnp.float32)

    hc = jnp.maximum(acc + gcb_ref[...], jnp.float32(0.0))
    out_ref[...] = jnp.dot(hc, fc_ref[...].T,
                           preferred_element_type=jnp.float32) + fcb_ref[...]


def _full(shape):
    nd = len(shape)
    return pl.BlockSpec(shape, lambda i: (0,) * nd)


def kernel(x, Win, emb_w, wc0_w, wc0_b, wc1_w, wc1_b, wc2_w, wc2_b,
           gc0_w, gc0_b, fc_w, fc_b, g_size):
    b0 = wc0_b.reshape(1, _F)
    b1 = wc1_b.reshape(1, _F)
    b2 = wc2_b.reshape(1, _F)
    gcb = gc0_b.reshape(1, _F)
    fcb = fc_b.reshape(1, _F)

    hF = jax.ShapeDtypeStruct((_N, _F), jnp.float32)
    hT = jax.ShapeDtypeStruct((_F, _N), jnp.float32)
    row_spec = pl.BlockSpec((_BLK, _F), lambda i: (i, 0))
    col_spec = pl.BlockSpec((_F, _BLK), lambda i: (0, i))
    outs = pl.pallas_call(
        _proj_body,
        grid=(_NBLK,),
        in_specs=[
            row_spec,
            _full((_F, _F)),
            _full((_F, _F)), _full((1, _F)),
            _full((_F, _F)), _full((1, _F)),
            _full((_F, _F)), _full((1, _F)),
            _full((_F, 7 * _F)),
        ],
        out_specs=[row_spec] * 3 + [col_spec] * 3 + [row_spec] * 7,
        out_shape=[hF] * 3 + [hT] * 3 + [hF] * 7,
        compiler_params=pltpu.CompilerParams(
            dimension_semantics=("parallel",)),
    )(x, emb_w, wc0_w, b0, wc1_w, b1, wc2_w, b2, gc0_w)

    out = pl.pallas_call(
        _main_body,
        grid=(_NBLK,),
        in_specs=[
            _full((_N, _F))] * 3 + [_full((_F, _N))] * 3 +
            [_full((_N, _F))] * 7 + [
            pl.BlockSpec((_BLK, _N), lambda i: (i, 0)),
            _full((1, _F)),
            _full((_F, _F)), _full((1, _F)),
        ],
        out_specs=pl.BlockSpec((_BLK, _F), lambda i: (i, 0)),
        out_shape=jax.ShapeDtypeStruct((_N, _F), jnp.float32),
        compiler_params=pltpu.CompilerParams(
            dimension_semantics=("parallel",)),
    )(*outs, Win, gcb, fc_w, fcb)

    return (out, Win, g_size)


# speedup vs baseline: 2.1755x; 1.0433x over previous
import jax
import jax.numpy as jnp
from jax.experimental import pallas as pl
from jax.experimental.pallas import tpu as pltpu

_N = 2048
_F = 128
_BLK = 256
_NBLK = _N // _BLK


def _proj_body(x_ref, emb_ref, w0_ref, b0_ref, w1_ref, b1_ref, w2_ref, b2_ref,
               gc_ref, h0_ref, h1_ref, h2_ref, h0t_ref, h1t_ref, h2t_ref,
               hg0_ref, hg1_ref, hg2_ref, hg3_ref, hg4_ref, hg5_ref, hg6_ref):
    h = jnp.dot(x_ref[...], emb_ref[...].T, preferred_element_type=jnp.float32)
    for k, (w_ref, b_ref, hk_ref, hkt_ref) in enumerate((
            (w0_ref, b0_ref, h0_ref, h0t_ref),
            (w1_ref, b1_ref, h1_ref, h1t_ref),
            (w2_ref, b2_ref, h2_ref, h2t_ref))):
        hk = jnp.dot(h, w_ref[...].T,
                     preferred_element_type=jnp.float32) + b_ref[...]
        hk_ref[...] = hk.astype(jnp.bfloat16)
        hkt_ref[...] = hk.T.astype(jnp.bfloat16)
    for c, hg_ref in enumerate((hg0_ref, hg1_ref, hg2_ref, hg3_ref,
                                hg4_ref, hg5_ref, hg6_ref)):
        hg = jnp.dot(h, gc_ref[:, c * _F:(c + 1) * _F].T,
                     preferred_element_type=jnp.float32)
        hg_ref[...] = hg if c == 0 else hg.astype(jnp.bfloat16)


def _main_body(h0_ref, h1_ref, h2_ref, h0t_ref, h1t_ref, h2t_ref,
               hg0_ref, hg1_ref, hg2_ref, hg3_ref, hg4_ref, hg5_ref, hg6_ref,
               win_ref, gcb_ref, fc_ref, fcb_ref, out_ref):
    i = pl.program_id(0)
    row = pl.ds(i * _BLK, _BLK)
    w1 = win_ref[...]
    w2 = w1 * w1
    w3 = w2 * w1
    scale = jnp.float32(1.0) / jnp.sqrt(jnp.float32(_F))

    acc = hg0_ref[row, :]

    for hk_ref, hkt_ref, wk, hga_ref, hgb_ref in (
            (h0_ref, h0t_ref, w1, hg1_ref, hg2_ref),
            (h1_ref, h1t_ref, w2, hg3_ref, hg4_ref),
            (h2_ref, h2t_ref, w3, hg5_ref, hg6_ref)):
        s = jax.nn.sigmoid(
            jnp.dot(hk_ref[row, :], hkt_ref[...],
                    preferred_element_type=jnp.float32) * scale)
        t = wk * s
        acc = acc + jnp.dot(t.astype(jnp.bfloat16), hga_ref[...],
                            preferred_element_type=jnp.float32)
        acc = acc + jnp.dot((t * s).astype(jnp.bfloat16), hgb_ref[...],
                            preferred_element_type=jnp.float32)

    hc = jnp.maximum(acc + gcb_ref[...], jnp.float32(0.0))
    out_ref[...] = jnp.dot(hc, fc_ref[...].T,
                           preferred_element_type=jnp.float32) + fcb_ref[...]


def _full(shape):
    nd = len(shape)
    return pl.BlockSpec(shape, lambda i: (0,) * nd)


def kernel(x, Win, emb_w, wc0_w, wc0_b, wc1_w, wc1_b, wc2_w, wc2_b,
           gc0_w, gc0_b, fc_w, fc_b, g_size):
    b0 = wc0_b.reshape(1, _F)
    b1 = wc1_b.reshape(1, _F)
    b2 = wc2_b.reshape(1, _F)
    gcb = gc0_b.reshape(1, _F)
    fcb = fc_b.reshape(1, _F)

    hF = jax.ShapeDtypeStruct((_N, _F), jnp.float32)
    hFb = jax.ShapeDtypeStruct((_N, _F), jnp.bfloat16)
    hT = jax.ShapeDtypeStruct((_F, _N), jnp.bfloat16)
    row_spec = pl.BlockSpec((_BLK, _F), lambda i: (i, 0))
    col_spec = pl.BlockSpec((_F, _BLK), lambda i: (0, i))
    outs = pl.pallas_call(
        _proj_body,
        grid=(_NBLK,),
        in_specs=[
            row_spec,
            _full((_F, _F)),
            _full((_F, _F)), _full((1, _F)),
            _full((_F, _F)), _full((1, _F)),
            _full((_F, _F)), _full((1, _F)),
            _full((_F, 7 * _F)),
        ],
        out_specs=[row_spec] * 3 + [col_spec] * 3 + [row_spec] * 7,
        out_shape=[hFb] * 3 + [hT] * 3 + [hF] + [hFb] * 6,
        compiler_params=pltpu.CompilerParams(
            dimension_semantics=("parallel",)),
    )(x, emb_w, wc0_w, b0, wc1_w, b1, wc2_w, b2, gc0_w)

    out = pl.pallas_call(
        _main_body,
        grid=(_NBLK,),
        in_specs=[
            _full((_N, _F))] * 3 + [_full((_F, _N))] * 3 +
            [_full((_N, _F))] * 7 + [
            pl.BlockSpec((_BLK, _N), lambda i: (i, 0)),
            _full((1, _F)),
            _full((_F, _F)), _full((1, _F)),
        ],
        out_specs=pl.BlockSpec((_BLK, _F), lambda i: (i, 0)),
        out_shape=jax.ShapeDtypeStruct((_N, _F), jnp.float32),
        compiler_params=pltpu.CompilerParams(
            dimension_semantics=("parallel",)),
    )(*outs, Win, gcb, fc_w, fcb)

    return (out, Win, g_size)


# speedup vs baseline: 2.5596x; 1.1765x over previous
import jax
import jax.numpy as jnp
from jax.experimental import pallas as pl
from jax.experimental.pallas import tpu as pltpu

_N = 2048
_F = 128
_BLK = 256
_NBLK = _N // _BLK


def _body(x_ref, emb_ref, w0_ref, b0_ref, w1_ref, b1_ref, w2_ref, b2_ref,
          gc_ref, gcb_ref, fc_ref, fcb_ref, win_ref, out_ref,
          h0_s, h1_s, h2_s, h0t_s, h1t_s, h2t_s,
          hg0_s, hg1_s, hg2_s, hg3_s, hg4_s, hg5_s, hg6_s):
    i = pl.program_id(0)

    @pl.when(i == 0)
    def _init():
        h = jnp.dot(x_ref[...], emb_ref[...].T,
                    preferred_element_type=jnp.float32)
        for w_ref, b_ref, hk_s, hkt_s in (
                (w0_ref, b0_ref, h0_s, h0t_s),
                (w1_ref, b1_ref, h1_s, h1t_s),
                (w2_ref, b2_ref, h2_s, h2t_s)):
            hk = jnp.dot(h, w_ref[...].T,
                         preferred_element_type=jnp.float32) + b_ref[...]
            hk_s[...] = hk.astype(jnp.bfloat16)
            hkt_s[...] = hk.T.astype(jnp.bfloat16)
        for c, hg_s in enumerate((hg0_s, hg1_s, hg2_s, hg3_s,
                                  hg4_s, hg5_s, hg6_s)):
            hg = jnp.dot(h, gc_ref[:, c * _F:(c + 1) * _F].T,
                         preferred_element_type=jnp.float32)
            hg_s[...] = hg if c == 0 else hg.astype(jnp.bfloat16)

    row = pl.ds(i * _BLK, _BLK)
    w1 = win_ref[...]
    w2 = w1 * w1
    w3 = w2 * w1
    scale = jnp.float32(1.0) / jnp.sqrt(jnp.float32(_F))

    acc = hg0_s[row, :]

    for hk_s, hkt_s, wk, hga_s, hgb_s in (
            (h0_s, h0t_s, w1, hg1_s, hg2_s),
            (h1_s, h1t_s, w2, hg3_s, hg4_s),
            (h2_s, h2t_s, w3, hg5_s, hg6_s)):
        s = jax.nn.sigmoid(
            jnp.dot(hk_s[row, :], hkt_s[...],
                    preferred_element_type=jnp.float32) * scale)
        t = wk * s
        acc = acc + jnp.dot(t.astype(jnp.bfloat16), hga_s[...],
                            preferred_element_type=jnp.float32)
        acc = acc + jnp.dot((t * s).astype(jnp.bfloat16), hgb_s[...],
                            preferred_element_type=jnp.float32)

    hc = jnp.maximum(acc + gcb_ref[...], jnp.float32(0.0))
    out_ref[...] = jnp.dot(hc, fc_ref[...].T,
                           preferred_element_type=jnp.float32) + fcb_ref[...]


def _full(shape):
    nd = len(shape)
    return pl.BlockSpec(shape, lambda i: (0,) * nd)


def kernel(x, Win, emb_w, wc0_w, wc0_b, wc1_w, wc1_b, wc2_w, wc2_b,
           gc0_w, gc0_b, fc_w, fc_b, g_size):
    b0 = wc0_b.reshape(1, _F)
    b1 = wc1_b.reshape(1, _F)
    b2 = wc2_b.reshape(1, _F)
    gcb = gc0_b.reshape(1, _F)
    fcb = fc_b.reshape(1, _F)

    bf = jnp.bfloat16
    scratch = (
        [pltpu.VMEM((_N, _F), bf)] * 3 +
        [pltpu.VMEM((_F, _N), bf)] * 3 +
        [pltpu.VMEM((_N, _F), jnp.float32)] +
        [pltpu.VMEM((_N, _F), bf)] * 6
    )

    out = pl.pallas_call(
        _body,
        grid=(_NBLK,),
        in_specs=[
            _full((_N, _F)),
            _full((_F, _F)),
            _full((_F, _F)), _full((1, _F)),
            _full((_F, _F)), _full((1, _F)),
            _full((_F, _F)), _full((1, _F)),
            _full((_F, 7 * _F)), _full((1, _F)),
            _full((_F, _F)), _full((1, _F)),
            pl.BlockSpec((_BLK, _N), lambda i: (i, 0)),
        ],
        out_specs=pl.BlockSpec((_BLK, _F), lambda i: (i, 0)),
        out_shape=jax.ShapeDtypeStruct((_N, _F), jnp.float32),
        scratch_shapes=scratch,
    )(x, emb_w, wc0_w, b0, wc1_w, b1, wc2_w, b2, gc0_w, gcb, fc_w, fcb, Win)

    return (out, Win, g_size)


# speedup vs baseline: 2.5762x; 1.0065x over previous
import jax
import jax.numpy as jnp
from jax.experimental import pallas as pl
from jax.experimental.pallas import tpu as pltpu

_N = 2048
_F = 128
_BLK = 512
_NBLK = _N // _BLK


def _body(x_ref, emb_ref, w0_ref, b0_ref, w1_ref, b1_ref, w2_ref, b2_ref,
          gc_ref, gcb_ref, fc_ref, fcb_ref, win_ref, out_ref,
          h0_s, h1_s, h2_s, h0t_s, h1t_s, h2t_s,
          hg0_s, hg1_s, hg2_s, hg3_s, hg4_s, hg5_s, hg6_s):
    i = pl.program_id(0)

    @pl.when(i == 0)
    def _init():
        h = jnp.dot(x_ref[...], emb_ref[...].T,
                    preferred_element_type=jnp.float32)
        for w_ref, b_ref, hk_s, hkt_s in (
                (w0_ref, b0_ref, h0_s, h0t_s),
                (w1_ref, b1_ref, h1_s, h1t_s),
                (w2_ref, b2_ref, h2_s, h2t_s)):
            hk = jnp.dot(h, w_ref[...].T,
                         preferred_element_type=jnp.float32) + b_ref[...]
            hk_s[...] = hk.astype(jnp.bfloat16)
            hkt_s[...] = hk.T.astype(jnp.bfloat16)
        for c, hg_s in enumerate((hg0_s, hg1_s, hg2_s, hg3_s,
                                  hg4_s, hg5_s, hg6_s)):
            hg = jnp.dot(h, gc_ref[:, c * _F:(c + 1) * _F].T,
                         preferred_element_type=jnp.float32)
            hg_s[...] = hg if c == 0 else hg.astype(jnp.bfloat16)

    row = pl.ds(i * _BLK, _BLK)
    w1 = win_ref[...]
    w2 = w1 * w1
    w3 = w2 * w1
    scale = jnp.float32(1.0) / jnp.sqrt(jnp.float32(_F))

    acc = hg0_s[row, :]

    for hk_s, hkt_s, wk, hga_s, hgb_s in (
            (h0_s, h0t_s, w1, hg1_s, hg2_s),
            (h1_s, h1t_s, w2, hg3_s, hg4_s),
            (h2_s, h2t_s, w3, hg5_s, hg6_s)):
        s = jax.nn.sigmoid(
            jnp.dot(hk_s[row, :], hkt_s[...],
                    preferred_element_type=jnp.float32) * scale)
        t = wk * s
        acc = acc + jnp.dot(t.astype(jnp.bfloat16), hga_s[...],
                            preferred_element_type=jnp.float32)
        acc = acc + jnp.dot((t * s).astype(jnp.bfloat16), hgb_s[...],
                            preferred_element_type=jnp.float32)

    hc = jnp.maximum(acc + gcb_ref[...], jnp.float32(0.0))
    out_ref[...] = jnp.dot(hc, fc_ref[...].T,
                           preferred_element_type=jnp.float32) + fcb_ref[...]


def _full(shape):
    nd = len(shape)
    return pl.BlockSpec(shape, lambda i: (0,) * nd)


def kernel(x, Win, emb_w, wc0_w, wc0_b, wc1_w, wc1_b, wc2_w, wc2_b,
           gc0_w, gc0_b, fc_w, fc_b, g_size):
    b0 = wc0_b.reshape(1, _F)
    b1 = wc1_b.reshape(1, _F)
    b2 = wc2_b.reshape(1, _F)
    gcb = gc0_b.reshape(1, _F)
    fcb = fc_b.reshape(1, _F)

    bf = jnp.bfloat16
    scratch = (
        [pltpu.VMEM((_N, _F), bf)] * 3 +
        [pltpu.VMEM((_F, _N), bf)] * 3 +
        [pltpu.VMEM((_N, _F), jnp.float32)] +
        [pltpu.VMEM((_N, _F), bf)] * 6
    )

    out = pl.pallas_call(
        _body,
        grid=(_NBLK,),
        in_specs=[
            _full((_N, _F)),
            _full((_F, _F)),
            _full((_F, _F)), _full((1, _F)),
            _full((_F, _F)), _full((1, _F)),
            _full((_F, _F)), _full((1, _F)),
            _full((_F, 7 * _F)), _full((1, _F)),
            _full((_F, _F)), _full((1, _F)),
            pl.BlockSpec((_BLK, _N), lambda i: (i, 0)),
        ],
        out_specs=pl.BlockSpec((_BLK, _F), lambda i: (i, 0)),
        out_shape=jax.ShapeDtypeStruct((_N, _F), jnp.float32),
        scratch_shapes=scratch,
    )(x, emb_w, wc0_w, b0, wc1_w, b1, wc2_w, b2, gc0_w, gcb, fc_w, fcb, Win)

    return (out, Win, g_size)


# speedup vs baseline: 2.6214x; 1.0175x over previous
import jax
import jax.numpy as jnp
from jax.experimental import pallas as pl
from jax.experimental.pallas import tpu as pltpu

_N = 2048
_F = 128
_BLK = 512
_NBLK = _N // _BLK


def _body(x_ref, emb_ref, w0_ref, b0_ref, w1_ref, b1_ref, w2_ref, b2_ref,
          gc_ref, gcb_ref, fc_ref, fcb_ref, win_ref, out_ref,
          h0_s, h1_s, h2_s, h0t_s, h1t_s, h2t_s,
          hg0_s, hg1_s, hg2_s, hg3_s, hg4_s, hg5_s, hg6_s):
    i = pl.program_id(0)
    scale = jnp.float32(1.0) / jnp.sqrt(jnp.float32(_F))

    @pl.when(i == 0)
    def _init():
        h = jnp.dot(x_ref[...], emb_ref[...].T,
                    preferred_element_type=jnp.float32)
        for w_ref, b_ref, hk_s, hkt_s in (
                (w0_ref, b0_ref, h0_s, h0t_s),
                (w1_ref, b1_ref, h1_s, h1t_s),
                (w2_ref, b2_ref, h2_s, h2t_s)):
            hk = jnp.dot(h, w_ref[...].T,
                         preferred_element_type=jnp.float32) + b_ref[...]
            hk_s[...] = hk.astype(jnp.bfloat16)
            hkt_s[...] = (hk.T * scale).astype(jnp.bfloat16)
        for c, hg_s in enumerate((hg0_s, hg1_s, hg2_s, hg3_s,
                                  hg4_s, hg5_s, hg6_s)):
            hg = jnp.dot(h, gc_ref[:, c * _F:(c + 1) * _F].T,
                         preferred_element_type=jnp.float32)
            hg_s[...] = hg if c == 0 else hg.astype(jnp.bfloat16)

    row = pl.ds(i * _BLK, _BLK)
    w1 = win_ref[...].astype(jnp.bfloat16)
    w2 = w1 * w1
    w3 = w2 * w1

    acc = hg0_s[row, :]

    for hk_s, hkt_s, wk, hga_s, hgb_s in (
            (h0_s, h0t_s, w1, hg1_s, hg2_s),
            (h1_s, h1t_s, w2, hg3_s, hg4_s),
            (h2_s, h2t_s, w3, hg5_s, hg6_s)):
        s = jax.nn.sigmoid(
            jnp.dot(hk_s[row, :], hkt_s[...],
                    preferred_element_type=jnp.float32))
        sb = s.astype(jnp.bfloat16)
        t = wk * sb
        acc = acc + jnp.dot(t, hga_s[...],
                            preferred_element_type=jnp.float32)
        acc = acc + jnp.dot(t * sb, hgb_s[...],
                            preferred_element_type=jnp.float32)

    hc = jnp.maximum(acc + gcb_ref[...], jnp.float32(0.0))
    out_ref[...] = jnp.dot(hc, fc_ref[...].T,
                           preferred_element_type=jnp.float32) + fcb_ref[...]


def _full(shape):
    nd = len(shape)
    return pl.BlockSpec(shape, lambda i: (0,) * nd)


def kernel(x, Win, emb_w, wc0_w, wc0_b, wc1_w, wc1_b, wc2_w, wc2_b,
           gc0_w, gc0_b, fc_w, fc_b, g_size):
    b0 = wc0_b.reshape(1, _F)
    b1 = wc1_b.reshape(1, _F)
    b2 = wc2_b.reshape(1, _F)
    gcb = gc0_b.reshape(1, _F)
    fcb = fc_b.reshape(1, _F)

    bf = jnp.bfloat16
    scratch = (
        [pltpu.VMEM((_N, _F), bf)] * 3 +
        [pltpu.VMEM((_F, _N), bf)] * 3 +
        [pltpu.VMEM((_N, _F), jnp.float32)] +
        [pltpu.VMEM((_N, _F), bf)] * 6
    )

    out = pl.pallas_call(
        _body,
        grid=(_NBLK,),
        in_specs=[
            _full((_N, _F)),
            _full((_F, _F)),
            _full((_F, _F)), _full((1, _F)),
            _full((_F, _F)), _full((1, _F)),
            _full((_F, _F)), _full((1, _F)),
            _full((_F, 7 * _F)), _full((1, _F)),
            _full((_F, _F)), _full((1, _F)),
            pl.BlockSpec((_BLK, _N), lambda i: (i, 0)),
        ],
        out_specs=pl.BlockSpec((_BLK, _F), lambda i: (i, 0)),
        out_shape=jax.ShapeDtypeStruct((_N, _F), jnp.float32),
        scratch_shapes=scratch,
    )(x, emb_w, wc0_w, b0, wc1_w, b1, wc2_w, b2, gc0_w, gcb, fc_w, fcb, Win)

    return (out, Win, g_size)


# speedup vs baseline: 3.3639x; 1.2833x over previous
import jax
import jax.numpy as jnp
from jax.experimental import pallas as pl
from jax.experimental.pallas import tpu as pltpu

_N = 2048
_F = 128
_BLK = 512
_NBLK = _N // _BLK


def _body(x_ref, emb_ref, w0_ref, b0_ref, w1_ref, b1_ref, w2_ref, b2_ref,
          gc_ref, gcb_ref, fc_ref, fcb_ref, win_ref, out_ref, win_out_ref,
          h0_s, h1_s, h2_s, h0t_s, h1t_s, h2t_s,
          hg0_s, hg1_s, hg2_s, hg3_s, hg4_s, hg5_s, hg6_s):
    i = pl.program_id(0)
    scale = jnp.float32(1.0) / jnp.sqrt(jnp.float32(_F))

    @pl.when(i == 0)
    def _init():
        h = jnp.dot(x_ref[...], emb_ref[...].T,
                    preferred_element_type=jnp.float32)
        for w_ref, b_ref, hk_s, hkt_s in (
                (w0_ref, b0_ref, h0_s, h0t_s),
                (w1_ref, b1_ref, h1_s, h1t_s),
                (w2_ref, b2_ref, h2_s, h2t_s)):
            hk = jnp.dot(h, w_ref[...].T,
                         preferred_element_type=jnp.float32) + b_ref[...]
            hk_s[...] = hk.astype(jnp.bfloat16)
            hkt_s[...] = (hk.T * scale).astype(jnp.bfloat16)
        for c, hg_s in enumerate((hg0_s, hg1_s, hg2_s, hg3_s,
                                  hg4_s, hg5_s, hg6_s)):
            hg = jnp.dot(h, gc_ref[:, c * _F:(c + 1) * _F].T,
                         preferred_element_type=jnp.float32)
            hg_s[...] = hg if c == 0 else hg.astype(jnp.bfloat16)

    row = pl.ds(i * _BLK, _BLK)
    win_out_ref[...] = win_ref[...]
    w1 = win_ref[...].astype(jnp.bfloat16)
    w2 = w1 * w1
    w3 = w2 * w1

    acc = hg0_s[row, :]

    for hk_s, hkt_s, wk, hga_s, hgb_s in (
            (h0_s, h0t_s, w1, hg1_s, hg2_s),
            (h1_s, h1t_s, w2, hg3_s, hg4_s),
            (h2_s, h2t_s, w3, hg5_s, hg6_s)):
        s = jax.nn.sigmoid(
            jnp.dot(hk_s[row, :], hkt_s[...],
                    preferred_element_type=jnp.float32))
        sb = s.astype(jnp.bfloat16)
        t = wk * sb
        acc = acc + jnp.dot(t, hga_s[...],
                            preferred_element_type=jnp.float32)
        acc = acc + jnp.dot(t * sb, hgb_s[...],
                            preferred_element_type=jnp.float32)

    hc = jnp.maximum(acc + gcb_ref[...], jnp.float32(0.0))
    out_ref[...] = jnp.dot(hc, fc_ref[...].T,
                           preferred_element_type=jnp.float32) + fcb_ref[...]


def _full(shape):
    nd = len(shape)
    return pl.BlockSpec(shape, lambda i: (0,) * nd)


def kernel(x, Win, emb_w, wc0_w, wc0_b, wc1_w, wc1_b, wc2_w, wc2_b,
           gc0_w, gc0_b, fc_w, fc_b, g_size):
    b0 = wc0_b.reshape(1, _F)
    b1 = wc1_b.reshape(1, _F)
    b2 = wc2_b.reshape(1, _F)
    gcb = gc0_b.reshape(1, _F)
    fcb = fc_b.reshape(1, _F)

    bf = jnp.bfloat16
    scratch = (
        [pltpu.VMEM((_N, _F), bf)] * 3 +
        [pltpu.VMEM((_F, _N), bf)] * 3 +
        [pltpu.VMEM((_N, _F), jnp.float32)] +
        [pltpu.VMEM((_N, _F), bf)] * 6
    )

    out = pl.pallas_call(
        _body,
        grid=(_NBLK,),
        in_specs=[
            _full((_N, _F)),
            _full((_F, _F)),
            _full((_F, _F)), _full((1, _F)),
            _full((_F, _F)), _full((1, _F)),
            _full((_F, _F)), _full((1, _F)),
            _full((_F, 7 * _F)), _full((1, _F)),
            _full((_F, _F)), _full((1, _F)),
            pl.BlockSpec((_BLK, _N), lambda i: (i, 0)),
        ],
        out_specs=[pl.BlockSpec((_BLK, _F), lambda i: (i, 0)),
                   pl.BlockSpec((_BLK, _N), lambda i: (i, 0))],
        out_shape=[jax.ShapeDtypeStruct((_N, _F), jnp.float32),
                   jax.ShapeDtypeStruct((_N, _N), jnp.float32)],
        scratch_shapes=scratch,
    )(x, emb_w, wc0_w, b0, wc1_w, b1, wc2_w, b2, gc0_w, gcb, fc_w, fcb, Win)

    out, win_out = out

    return (out, win_out, g_size)
